# baseline (speedup 1.0000x reference)
import jax
import jax.numpy as jnp
from jax import lax
from jax.experimental import pallas as pl
from jax.experimental.pallas import tpu as pltpu
from jax.experimental.pallas import tpu_sc as plsc

_B = 4096
_E = 32
_NF = 8

_info = plsc.get_sparse_core_info()
_NC, _NS = _info.num_cores, _info.num_subcores
_NW = _NC * _NS
_BPW = _B // _NW


def _sc_gather_body(*refs):
    idx_refs = refs[0:_NF]
    tab_refs = refs[_NF:2 * _NF]
    user_out, item_out = refs[2 * _NF], refs[2 * _NF + 1]
    idx_v, rows_v, sem = refs[2 * _NF + 2], refs[2 * _NF + 3], refs[2 * _NF + 4]

    wid = lax.axis_index("s") * _NC + lax.axis_index("c")
    base = wid * _BPW

    for f in range(_NF):
        pltpu.sync_copy(idx_refs[f].at[pl.ds(base, _BPW)], idx_v.at[f])
    copies = []
    for f in range(_NF):
        copies.append(
            pltpu.async_copy(tab_refs[f].at[idx_v.at[f]], rows_v.at[f], sem))
    for c in copies:
        c.wait()
    for f in range(4):
        pltpu.sync_copy(rows_v.at[f], user_out.at[pl.ds(base, _BPW), f])
    for f in range(4):
        pltpu.sync_copy(rows_v.at[4 + f], item_out.at[pl.ds(base, _BPW), f])


_sc_gather = pl.kernel(
    _sc_gather_body,
    out_type=[jax.ShapeDtypeStruct((_B, 4, _E), jnp.float32)] * 2,
    mesh=plsc.VectorSubcoreMesh(core_axis_name="c", subcore_axis_name="s"),
    scratch_types=[
        pltpu.VMEM((_NF, _BPW), jnp.int32),
        pltpu.VMEM((_NF, _BPW, _E), jnp.float32),
        pltpu.SemaphoreType.DMA,
    ],
    compiler_params=pltpu.CompilerParams(use_tc_tiling_on_sc=False),
)


def _tower_body(ue, ie, uw1, ub1, uw2, ub2, iw1, ib1, iw2, ib2, out):
    u = jnp.maximum(
        jnp.dot(ue[...], uw1[...], preferred_element_type=jnp.float32)
        + ub1[...], 0.0)
    u = jnp.maximum(
        jnp.dot(u, uw2[...], preferred_element_type=jnp.float32)
        + ub2[...], 0.0)
    v = jnp.maximum(
        jnp.dot(ie[...], iw1[...], preferred_element_type=jnp.float32)
        + ib1[...], 0.0)
    v = jnp.maximum(
        jnp.dot(v, iw2[...], preferred_element_type=jnp.float32)
        + ib2[...], 0.0)
    s_ui = jnp.sum(u * v)
    s_uu = jnp.sum(u * u)
    s_ii = jnp.sum(v * v)
    out[0, 0] = s_ui / jnp.sqrt(s_uu * s_ii)


_tower = pl.pallas_call(
    _tower_body,
    out_shape=jax.ShapeDtypeStruct((1, 1), jnp.float32),
    in_specs=[pl.BlockSpec(memory_space=pltpu.VMEM)] * 10,
    out_specs=pl.BlockSpec(memory_space=pltpu.SMEM),
)


def kernel(idx_u0, idx_u1, idx_u2, idx_u3, idx_i0, idx_i1, idx_i2, idx_i3,
           table_u0, table_u1, table_u2, table_u3,
           table_i0, table_i1, table_i2, table_i3,
           user_W1, user_b1, user_W2, user_b2,
           item_W1, item_b1, item_W2, item_b2):
    idxs = [x.reshape(-1).astype(jnp.int32)
            for x in (idx_u0, idx_u1, idx_u2, idx_u3,
                      idx_i0, idx_i1, idx_i2, idx_i3)]
    tabs = [table_u0, table_u1, table_u2, table_u3,
            table_i0, table_i1, table_i2, table_i3]
    user_e, item_e = _sc_gather(*idxs, *tabs)
    out = _tower(user_e.reshape(_B, 4 * _E), item_e.reshape(_B, 4 * _E),
                 user_W1, user_b1.reshape(1, 64), user_W2,
                 user_b2.reshape(1, 32),
                 item_W1, item_b1.reshape(1, 64), item_W2,
                 item_b2.reshape(1, 32))
    return out[0, 0]
